# baseline (speedup 1.0000x reference)
import functools

import jax
import jax.numpy as jnp
from jax import lax
from jax.experimental import pallas as pl
from jax.experimental.pallas import tpu as pltpu
from jax.experimental.pallas import tpu_sc as plsc

B, F, V, D = 4096, 26, 100000, 16
DH = D // 2
NC, NS = 2, 16
NW = NC * NS
BPW = B // NW
NCHUNK = BPW // 16

_mesh = plsc.VectorSubcoreMesh(core_axis_name="c", subcore_axis_name="s")
_params = pltpu.CompilerParams(
    needs_layout_passes=False, use_tc_tiling_on_sc=False)


def _gather_half(idx_hbm, tab_hbm, wid, idx_v, emb_v, acc_v, acc2_v,
                 sem_e, extra_fire=None, extra_wait=None):
    pltpu.sync_copy(idx_hbm.at[:, pl.ds(wid * BPW, BPW)], idx_v)

    def accumulate(f, init):
        def dbody(d, c, f=f, init=init):
            for s in range(NCHUNK):
                v = emb_v[f * DH + d, pl.ds(s * 16, 16)]
                if init:
                    acc_v[d, pl.ds(s * 16, 16)] = v
                    acc2_v[d, pl.ds(s * 16, 16)] = v * v
                else:
                    acc_v[d, pl.ds(s * 16, 16)] = (
                        acc_v[d, pl.ds(s * 16, 16)] + v)
                    acc2_v[d, pl.ds(s * 16, 16)] = (
                        acc2_v[d, pl.ds(s * 16, 16)] + v * v)
            return c
        lax.fori_loop(0, DH, dbody, 0)

    window = []
    done = 0
    for f in range(F):
        row = idx_v.at[f]
        fired = [pltpu.async_copy(
            tab_hbm.at[f, d].at[row], emb_v.at[f * DH + d], sem_e)
            for d in range(DH)]
        if extra_fire is not None:
            fired.append(extra_fire(f, row))
        window.append(fired)
        if len(window) > 3:
            for cp in window.pop(0):
                cp.wait()
            accumulate(done, done == 0)
            done += 1
    if extra_wait is not None:
        extra_wait()
    for fired in window:
        for cp in fired:
            cp.wait()
        accumulate(done, done == 0)
        done += 1


def _interaction_chunk(acc_v, acc2_v, s):
    inter = jnp.zeros((16,), jnp.float32)
    for d in range(DH):
        a = acc_v[d, pl.ds(s * 16, 16)]
        inter = inter + (a * a - acc2_v[d, pl.ds(s * 16, 16)])
    return inter


@functools.partial(
    pl.kernel, mesh=_mesh, compiler_params=_params,
    out_type=jax.ShapeDtypeStruct((B,), jnp.float32),
    scratch_types=[
        pltpu.VMEM((F, BPW), jnp.int32),
        pltpu.VMEM((F * DH, BPW), jnp.float32),
        pltpu.VMEM((DH, BPW), jnp.float32),
        pltpu.VMEM((DH, BPW), jnp.float32),
        pltpu.VMEM((BPW,), jnp.float32),
        pltpu.SemaphoreType.DMA,
    ],
)
def _fm_lo(idx_hbm, tab_hbm, out_hbm,
           idx_v, emb_v, acc_v, acc2_v, out_v, sem_e):
    wid = lax.axis_index("s") * NC + lax.axis_index("c")
    _gather_half(idx_hbm, tab_hbm, wid, idx_v, emb_v, acc_v, acc2_v, sem_e)
    for s in range(NCHUNK):
        out_v[pl.ds(s * 16, 16)] = _interaction_chunk(acc_v, acc2_v, s)
    pltpu.sync_copy(out_v, out_hbm.at[pl.ds(wid * BPW, BPW)])


@functools.partial(
    pl.kernel, mesh=_mesh, compiler_params=_params,
    out_type=jax.ShapeDtypeStruct((B,), jnp.float32),
    scratch_types=[
        pltpu.VMEM((F, BPW), jnp.int32),
        pltpu.VMEM((F * DH, BPW), jnp.float32),
        pltpu.VMEM((DH, BPW), jnp.float32),
        pltpu.VMEM((DH, BPW), jnp.float32),
        pltpu.VMEM((F, BPW), jnp.float32),
        pltpu.VMEM((BPW,), jnp.float32),
        pltpu.VMEM((BPW,), jnp.float32),
        pltpu.VMEM((16,), jnp.float32),
        pltpu.SemaphoreType.DMA,
        pltpu.SemaphoreType.DMA,
    ],
)
def _fm_hi(idx_hbm, tab_hbm, w_hbm, bias_hbm, lo_hbm, out_hbm,
           idx_v, emb_v, acc_v, acc2_v, w_v, lo_v, out_v, bias_v,
           sem_e, sem_w):
    wid = lax.axis_index("s") * NC + lax.axis_index("c")

    def fire_w(f, row):
        return pltpu.async_copy(w_hbm.at[f].at[row], w_v.at[f], sem_w)

    def wait_rest():
        pltpu.sync_copy(bias_hbm, bias_v)
        pltpu.sync_copy(lo_hbm.at[pl.ds(wid * BPW, BPW)], lo_v)

    _gather_half(idx_hbm, tab_hbm, wid, idx_v, emb_v, acc_v, acc2_v, sem_e,
                 extra_fire=fire_w, extra_wait=wait_rest)

    bias_vec = bias_v[...]
    for s in range(NCHUNK):
        inter = _interaction_chunk(acc_v, acc2_v, s)
        inter = inter + lo_v[pl.ds(s * 16, 16)]
        lin = bias_vec
        for f in range(F):
            lin = lin + w_v[f, pl.ds(s * 16, 16)]
        x = lin + 0.5 * inter
        out_v[pl.ds(s * 16, 16)] = 1.0 / (1.0 + jnp.exp(-x))

    pltpu.sync_copy(out_v, out_hbm.at[pl.ds(wid * BPW, BPW)])


def kernel(indices, tables, w_linear, bias):
    idx_t = indices.astype(jnp.int32).T
    tab_t = tables.transpose(0, 2, 1)
    bias_vec = jnp.broadcast_to(bias.astype(jnp.float32), (16,))
    lo = _fm_lo(idx_t, tab_t[:, :DH, :])
    return _fm_hi(idx_t, tab_t[:, DH:, :], w_linear, bias_vec, lo)


# speedup vs baseline: 1.2290x; 1.2290x over previous
import functools

import jax
import jax.numpy as jnp
from jax import lax
from jax.experimental import pallas as pl
from jax.experimental.pallas import tpu as pltpu
from jax.experimental.pallas import tpu_sc as plsc

B, F, V, D = 4096, 26, 100000, 16
NC, NS = 2, 16
NW = NC * NS
BPW = B // NW
NCHUNK = BPW // 16

_mesh = plsc.VectorSubcoreMesh(core_axis_name="c", subcore_axis_name="s")


@functools.partial(
    pl.kernel,
    mesh=_mesh,
    compiler_params=pltpu.CompilerParams(
        needs_layout_passes=False, use_tc_tiling_on_sc=False),
    out_type=jax.ShapeDtypeStruct((B,), jnp.float32),
    scratch_types=[
        pltpu.VMEM((F, BPW), jnp.int32),
        pltpu.VMEM((F * D, BPW), jnp.float32),
        pltpu.VMEM((F, BPW), jnp.float32),
        pltpu.VMEM((D, BPW), jnp.float32),
        pltpu.VMEM((D, BPW), jnp.float32),
        pltpu.VMEM((BPW,), jnp.float32),
        pltpu.VMEM((16,), jnp.float32),
        pltpu.SemaphoreType.DMA,
        pltpu.SemaphoreType.DMA,
    ],
)
def _fm_sc(idx_hbm, tab_hbm, w_hbm, bias_hbm, out_hbm,
           idx_v, emb_v, w_v, acc_v, acc2_v, out_v, bias_v, sem_e, sem_w):
    wid = lax.axis_index("s") * NC + lax.axis_index("c")
    pltpu.sync_copy(idx_hbm.at[:, pl.ds(wid * BPW, BPW)], idx_v)

    zeros = jnp.zeros((16,), jnp.float32)

    def accumulate(f, init):
        def dbody(d, c, f=f, init=init):
            for s in range(NCHUNK):
                v = emb_v[f * D + d, pl.ds(s * 16, 16)]
                if init:
                    acc_v[d, pl.ds(s * 16, 16)] = v
                    acc2_v[d, pl.ds(s * 16, 16)] = v * v
                else:
                    acc_v[d, pl.ds(s * 16, 16)] = (
                        acc_v[d, pl.ds(s * 16, 16)] + v)
                    acc2_v[d, pl.ds(s * 16, 16)] = (
                        acc2_v[d, pl.ds(s * 16, 16)] + v * v)
            return c
        lax.fori_loop(0, D, dbody, 0)

    window = []
    done = 0
    for f in range(F):
        row = idx_v.at[f]
        fired = [pltpu.async_copy(
            tab_hbm.at[f, d].at[row], emb_v.at[f * D + d], sem_e)
            for d in range(D)]
        fired.append(pltpu.async_copy(w_hbm.at[f].at[row], w_v.at[f], sem_w))
        window.append(fired)
        if len(window) > 3:
            for cp in window.pop(0):
                cp.wait()
            accumulate(done, done == 0)
            done += 1
    pltpu.sync_copy(bias_hbm, bias_v)
    for fired in window:
        for cp in fired:
            cp.wait()
        accumulate(done, done == 0)
        done += 1

    bias_vec = bias_v[...]
    for s in range(NCHUNK):
        inter = zeros
        for d in range(D):
            a = acc_v[d, pl.ds(s * 16, 16)]
            inter = inter + (a * a - acc2_v[d, pl.ds(s * 16, 16)])
        lin = bias_vec
        for f in range(F):
            lin = lin + w_v[f, pl.ds(s * 16, 16)]
        x = lin + 0.5 * inter
        out_v[pl.ds(s * 16, 16)] = 1.0 / (1.0 + jnp.exp(-x))

    pltpu.sync_copy(out_v, out_hbm.at[pl.ds(wid * BPW, BPW)])


def kernel(indices, tables, w_linear, bias):
    idx_t = indices.astype(jnp.int32).T
    tab_t = tables.transpose(0, 2, 1)
    bias_vec = jnp.broadcast_to(bias.astype(jnp.float32), (16,))
    return _fm_sc(idx_t, tab_t, w_linear, bias_vec)
